# Initial kernel scaffold; baseline (speedup 1.0000x reference)
import jax
import jax.numpy as jnp
from jax.experimental import pallas as pl


def kernel(x_flow, ei_src_of, ei_dst_of, ei_rev_src_of, ei_rev_dst_of, host_emb, c1_so_Wl, c1_so_bl, c1_so_Wr, c1_do_Wl, c1_do_bl, c1_do_Wr, c1_rs_Wl, c1_rs_bl, c1_rs_Wr, c1_rd_Wl, c1_rd_bl, c1_rd_Wr, c2_so_Wl, c2_so_bl, c2_so_Wr, c2_do_Wl, c2_do_bl, c2_do_Wr, c2_rs_Wl, c2_rs_bl, c2_rs_Wr, c2_rd_Wl, c2_rd_bl, c2_rd_Wr, n1f_g, n1f_b, n1h_g, n1h_b, n2f_g, n2f_b, n2h_g, n2h_b, lin_W, lin_b):
    raise NotImplementedError("write your pallas kernel here")


# speedup vs baseline: 2.6388x; 2.6388x over previous
import functools

import jax
import jax.numpy as jnp
from jax import lax
from jax.experimental import pallas as pl
from jax.experimental.pallas import tpu as pltpu
import jax.experimental.pallas.tpu_sc as plsc

NF = 100000
NH = 50000
DF = 128
EMB = 32
HID = 64
OUT = 2
EPS = 1e-5

NCORE = 2
NSUB = 16
NW = NCORE * NSUB
CH = 80
NCH = NF // CH
NHH = NH // NCORE
TRASH = 64
ZCH = 128

@functools.cache
def _mesh():
    return plsc.VectorSubcoreMesh(core_axis_name="c", subcore_axis_name="s",
                                  num_cores=NCORE, num_subcores=NSUB)


def _mm3_body(x_ref, w1_ref, w2_ref, w3_ref, b3_ref, o1_ref, o2_ref, o3_ref):
    x = x_ref[...]
    o1_ref[...] = jnp.dot(x, w1_ref[...], preferred_element_type=jnp.float32)
    o2_ref[...] = jnp.dot(x, w2_ref[...], preferred_element_type=jnp.float32)
    o3_ref[...] = (jnp.dot(x, w3_ref[...], preferred_element_type=jnp.float32)
                   + b3_ref[...])


def _mm3(x, w1, w2, w3, b3, br):
    n, d = x.shape
    grid = n // br
    return pl.pallas_call(
        _mm3_body,
        grid=(grid,),
        in_specs=[
            pl.BlockSpec((br, d), lambda i: (i, 0)),
            pl.BlockSpec((d, HID), lambda i: (0, 0)),
            pl.BlockSpec((d, HID), lambda i: (0, 0)),
            pl.BlockSpec((d, HID), lambda i: (0, 0)),
            pl.BlockSpec((1, HID), lambda i: (0, 0)),
        ],
        out_specs=[
            pl.BlockSpec((br, HID), lambda i: (i, 0)),
            pl.BlockSpec((br, HID), lambda i: (i, 0)),
            pl.BlockSpec((br, HID), lambda i: (i, 0)),
        ],
        out_shape=[jax.ShapeDtypeStruct((n, HID), jnp.float32)] * 3,
    )(x, w1, w2, w3, b3.reshape(1, HID))


def _stats_body(x_ref, s_ref, ss_ref):
    x = x_ref[...]
    ps = jnp.sum(x, axis=0, keepdims=True)
    pss = jnp.sum(x * x, axis=0, keepdims=True)

    @pl.when(pl.program_id(0) == 0)
    def _():
        s_ref[...] = ps
        ss_ref[...] = pss

    @pl.when(pl.program_id(0) != 0)
    def _():
        s_ref[...] += ps
        ss_ref[...] += pss


def _stats(x, br):
    n, _ = x.shape
    return pl.pallas_call(
        _stats_body,
        grid=(n // br,),
        in_specs=[pl.BlockSpec((br, HID), lambda i: (i, 0))],
        out_specs=[pl.BlockSpec((1, HID), lambda i: (0, 0))] * 2,
        out_shape=[jax.ShapeDtypeStruct((1, HID), jnp.float32)] * 2,
    )(x)


def _combine_body(ssrc_ref, sdst_ref, csrc_ref, cdst_ref, hr_ref, o_ref):
    c1 = jnp.maximum(csrc_ref[...][:, 0:1], 1.0)
    c2 = jnp.maximum(cdst_ref[...][:, 0:1], 1.0)
    o_ref[...] = ssrc_ref[...] / c1 + sdst_ref[...] / c2 + hr_ref[...]


def _combine(ssrc, sdst, csrc, cdst, hr, br):
    n, _ = ssrc.shape
    return pl.pallas_call(
        _combine_body,
        grid=(n // br,),
        in_specs=[
            pl.BlockSpec((br, HID), lambda i: (i, 0)),
            pl.BlockSpec((br, HID), lambda i: (i, 0)),
            pl.BlockSpec((br, 16), lambda i: (i, 0)),
            pl.BlockSpec((br, 16), lambda i: (i, 0)),
            pl.BlockSpec((br, HID), lambda i: (i, 0)),
        ],
        out_specs=pl.BlockSpec((br, HID), lambda i: (i, 0)),
        out_shape=jax.ShapeDtypeStruct((n, HID), jnp.float32),
    )(ssrc, sdst, csrc, cdst, hr)


def _make_bn_mm_body(n):
    def body(x_ref, s_ref, ss_ref, g_ref, b_ref, w_ref, bias_ref, o_ref):
        m = s_ref[...] / n
        v = ss_ref[...] / n - m * m
        scale = g_ref[...] * lax.rsqrt(v + EPS)
        shift = b_ref[...] - m * scale
        h = jnp.maximum(x_ref[...] * scale + shift, 0.0)
        o_ref[...] = (jnp.dot(h, w_ref[...], preferred_element_type=jnp.float32)
                      + bias_ref[...])
    return body


def _bn_mm(x, s, ss, g, b, w, bias, br):
    n, _ = x.shape
    k = w.shape[1]
    return pl.pallas_call(
        _make_bn_mm_body(float(n)),
        grid=(n // br,),
        in_specs=[
            pl.BlockSpec((br, HID), lambda i: (i, 0)),
            pl.BlockSpec((1, HID), lambda i: (0, 0)),
            pl.BlockSpec((1, HID), lambda i: (0, 0)),
            pl.BlockSpec((1, HID), lambda i: (0, 0)),
            pl.BlockSpec((1, HID), lambda i: (0, 0)),
            pl.BlockSpec((HID, k), lambda i: (0, 0)),
            pl.BlockSpec((1, k), lambda i: (0, 0)),
        ],
        out_specs=pl.BlockSpec((br, k), lambda i: (i, 0)),
        out_shape=jax.ShapeDtypeStruct((n, k), jnp.float32),
    )(x, s, ss, g.reshape(1, HID), b.reshape(1, HID), w, bias.reshape(1, k))


def _sc_gather_add_body(ta_h, tb_h, ia_h, ib_h, xc_h, out_h,
                        ia_v, ib_v, ra_v, rb_v, rc_v, sem_a, sem_b):
    wid = lax.axis_index("s") * NCORE + lax.axis_index("c")

    def chunk(j, carry):
        c = wid + NW * j

        @pl.when(c < NCH)
        def _():
            base = c * CH
            pltpu.sync_copy(ia_h.at[pl.ds(base, CH)], ia_v)
            pltpu.sync_copy(ib_h.at[pl.ds(base, CH)], ib_v)
            cp_a = pltpu.async_copy(ta_h.at[ia_v], ra_v, sem_a)
            cp_b = pltpu.async_copy(tb_h.at[ib_v], rb_v, sem_b)
            pltpu.sync_copy(xc_h.at[pl.ds(base, CH)], rc_v)
            cp_a.wait()
            cp_b.wait()

            def add_g(g, carry2):
                r = g // (HID // 16)
                col = (g % (HID // 16)) * 16
                acc = (ra_v[r, pl.ds(col, 16)] + rb_v[r, pl.ds(col, 16)]
                       + rc_v[r, pl.ds(col, 16)])
                ra_v[r, pl.ds(col, 16)] = acc
                return carry2

            lax.fori_loop(0, CH * (HID // 16), add_g, 0)
            pltpu.sync_copy(ra_v, out_h.at[pl.ds(base, CH)])

        return carry

    lax.fori_loop(0, (NCH + NW - 1) // NW, chunk, 0)


def _sc_gather_add(ta, tb, ia, ib, xc):
    return pl.kernel(
        _sc_gather_add_body,
        out_type=jax.ShapeDtypeStruct((NF, HID), jnp.float32),
        mesh=_mesh(),
        compiler_params=pltpu.CompilerParams(use_tc_tiling_on_sc=False),
        scratch_types=[
            pltpu.VMEM((CH,), jnp.int32),
            pltpu.VMEM((CH,), jnp.int32),
            pltpu.VMEM((CH, HID), jnp.float32),
            pltpu.VMEM((CH, HID), jnp.float32),
            pltpu.VMEM((CH, HID), jnp.float32),
            pltpu.SemaphoreType.DMA,
            pltpu.SemaphoreType.DMA,
        ],
    )(ta, tb, ia, ib, xc)


def _sc_sums_body(pa_h, pb_h, ia_h, ib_h, sa_h, sb_h,
                  idx_v, idx2_v, val_v, zrow_v, ssum_sh):
    cid = lax.axis_index("c")
    sid = lax.axis_index("s")
    hbase = cid * NHH
    zero16 = jnp.zeros((16,), jnp.float32)

    def init_zero(g, carry):
        zrow_v[g // (HID // 16), pl.ds((g % (HID // 16)) * 16, 16)] = zero16
        return carry

    lax.fori_loop(0, ZCH * (HID // 16), init_zero, 0)

    nzc = (NHH + TRASH + ZCH - 1) // ZCH
    nfc = (NHH + ZCH - 1) // ZCH

    for p_h, i_h, s_h in ((pa_h, ia_h, sa_h), (pb_h, ib_h, sb_h)):
        def zero_chunk(j, carry):
            c = sid + NSUB * j

            @pl.when(c < nzc)
            def _():
                b = jnp.minimum(c * ZCH, NHH + TRASH - ZCH)
                pltpu.sync_copy(zrow_v, ssum_sh.at[pl.ds(b, ZCH)])

            return carry

        lax.fori_loop(0, (nzc + NSUB - 1) // NSUB, zero_chunk, 0)
        plsc.subcore_barrier()

        def scat_chunk(j, carry, i_h=i_h, p_h=p_h):
            c = sid + NSUB * j

            @pl.when(c < NCH)
            def _():
                base = c * CH
                pltpu.sync_copy(i_h.at[pl.ds(base, CH)], idx_v)

                def xform(g, carry2):
                    v = idx_v[pl.ds(g * 16, 16)]
                    loc = v - hbase
                    ok = (loc >= 0) & (loc < NHH)
                    tr = NHH + (v & (TRASH - 1))
                    idx2_v[pl.ds(g * 16, 16)] = jnp.where(ok, loc, tr)
                    return carry2

                lax.fori_loop(0, CH // 16, xform, 0)
                pltpu.sync_copy(p_h.at[pl.ds(base, CH)], val_v)
                pltpu.sync_copy(val_v, ssum_sh.at[idx2_v], add=True)

            return carry

        lax.fori_loop(0, (NCH + NSUB - 1) // NSUB, scat_chunk, 0)
        plsc.subcore_barrier()

        def flush_chunk(j, carry, s_h=s_h):
            c = sid + NSUB * j

            @pl.when(c < nfc)
            def _():
                b = jnp.minimum(c * ZCH, NHH - ZCH)
                pltpu.sync_copy(ssum_sh.at[pl.ds(b, ZCH)],
                                s_h.at[pl.ds(hbase + b, ZCH)])

            return carry

        lax.fori_loop(0, (nfc + NSUB - 1) // NSUB, flush_chunk, 0)
        plsc.subcore_barrier()


def _sc_sums(pa, pb, ia, ib):
    return pl.kernel(
        _sc_sums_body,
        out_type=[
            jax.ShapeDtypeStruct((NH, HID), jnp.float32),
            jax.ShapeDtypeStruct((NH, HID), jnp.float32),
        ],
        mesh=_mesh(),
        compiler_params=pltpu.CompilerParams(use_tc_tiling_on_sc=False),
        scratch_types=[
            pltpu.VMEM((CH,), jnp.int32),
            pltpu.VMEM((CH,), jnp.int32),
            pltpu.VMEM((CH, HID), jnp.float32),
            pltpu.VMEM((ZCH, HID), jnp.float32),
            pltpu.VMEM_SHARED((NHH + TRASH, HID), jnp.float32),
        ],
    )(pa, pb, ia, ib)


def _sc_counts_body(ia_h, ib_h, ca_h, cb_h,
                    idx_v, idx2_v, ones_v, zcnt_v, cta_sh, ctb_sh):
    cid = lax.axis_index("c")
    sid = lax.axis_index("s")
    hbase = cid * NHH
    ones16 = jnp.ones((16,), jnp.float32)
    zero16 = jnp.zeros((16,), jnp.float32)

    def init_bufs(r, carry):
        ones_v[r, :] = ones16
        return carry

    lax.fori_loop(0, CH, init_bufs, 0)

    def init_zcnt(r, carry):
        zcnt_v[r, :] = zero16
        return carry

    lax.fori_loop(0, ZCH, init_zcnt, 0)

    nzc = (NHH + TRASH + ZCH - 1) // ZCH
    nfc = (NHH + ZCH - 1) // ZCH

    def zero_chunk(j, carry):
        c = sid + NSUB * j

        @pl.when(c < nzc)
        def _():
            b = jnp.minimum(c * ZCH, NHH + TRASH - ZCH)
            pltpu.sync_copy(zcnt_v, cta_sh.at[pl.ds(b, ZCH)])
            pltpu.sync_copy(zcnt_v, ctb_sh.at[pl.ds(b, ZCH)])

        return carry

    lax.fori_loop(0, (nzc + NSUB - 1) // NSUB, zero_chunk, 0)
    plsc.subcore_barrier()

    def scat_chunk(j, carry):
        c = sid + NSUB * j

        @pl.when(c < NCH)
        def _():
            base = c * CH
            for i_h, ct_sh in ((ia_h, cta_sh), (ib_h, ctb_sh)):
                pltpu.sync_copy(i_h.at[pl.ds(base, CH)], idx_v)

                def xform(g, carry2):
                    v = idx_v[pl.ds(g * 16, 16)]
                    loc = v - hbase
                    ok = (loc >= 0) & (loc < NHH)
                    tr = NHH + (v & (TRASH - 1))
                    idx2_v[pl.ds(g * 16, 16)] = jnp.where(ok, loc, tr)
                    return carry2

                lax.fori_loop(0, CH // 16, xform, 0)
                pltpu.sync_copy(ones_v, ct_sh.at[idx2_v], add=True)

        return carry

    lax.fori_loop(0, (NCH + NSUB - 1) // NSUB, scat_chunk, 0)
    plsc.subcore_barrier()

    def flush_chunk(j, carry):
        c = sid + NSUB * j

        @pl.when(c < nfc)
        def _():
            b = jnp.minimum(c * ZCH, NHH - ZCH)
            pltpu.sync_copy(cta_sh.at[pl.ds(b, ZCH)],
                            ca_h.at[pl.ds(hbase + b, ZCH)])
            pltpu.sync_copy(ctb_sh.at[pl.ds(b, ZCH)],
                            cb_h.at[pl.ds(hbase + b, ZCH)])

        return carry

    lax.fori_loop(0, (nfc + NSUB - 1) // NSUB, flush_chunk, 0)
    plsc.subcore_barrier()


def _sc_counts(ia, ib):
    return pl.kernel(
        _sc_counts_body,
        out_type=[
            jax.ShapeDtypeStruct((NH, 16), jnp.float32),
            jax.ShapeDtypeStruct((NH, 16), jnp.float32),
        ],
        mesh=_mesh(),
        compiler_params=pltpu.CompilerParams(use_tc_tiling_on_sc=False),
        scratch_types=[
            pltpu.VMEM((CH,), jnp.int32),
            pltpu.VMEM((CH,), jnp.int32),
            pltpu.VMEM((CH, 16), jnp.float32),
            pltpu.VMEM((ZCH, 16), jnp.float32),
            pltpu.VMEM_SHARED((NHH + TRASH, 16), jnp.float32),
            pltpu.VMEM_SHARED((NHH + TRASH, 16), jnp.float32),
        ],
    )(ia, ib)


def _sc_scatter(pa, pb, ia, ib):
    sa, sb = _sc_sums(pa, pb, ia, ib)
    ca, cb = _sc_counts(ia, ib)
    return sa, sb, ca, cb


def kernel(x_flow, ei_src_of, ei_dst_of, ei_rev_src_of, ei_rev_dst_of, host_emb,
           c1_so_Wl, c1_so_bl, c1_so_Wr, c1_do_Wl, c1_do_bl, c1_do_Wr,
           c1_rs_Wl, c1_rs_bl, c1_rs_Wr, c1_rd_Wl, c1_rd_bl, c1_rd_Wr,
           c2_so_Wl, c2_so_bl, c2_so_Wr, c2_do_Wl, c2_do_bl, c2_do_Wr,
           c2_rs_Wl, c2_rs_bl, c2_rs_Wr, c2_rd_Wl, c2_rd_bl, c2_rd_Wr,
           n1f_g, n1f_b, n1h_g, n1h_b, n2f_g, n2f_b, n2h_g, n2h_b,
           lin_W, lin_b):
    src = ei_src_of[0]
    dst = ei_dst_of[0]
    ssrc = ei_rev_src_of[1]
    sdst = ei_rev_dst_of[1]

    p_rs, p_rd, xr = _mm3(x_flow, c1_rs_Wl, c1_rd_Wl, c1_so_Wr + c1_do_Wr,
                          c1_so_bl + c1_do_bl, br=2000)
    hso, hdo, hr = _mm3(host_emb, c1_so_Wl, c1_do_Wl, c1_rs_Wr + c1_rd_Wr,
                        c1_rs_bl + c1_rd_bl, br=2000)

    f1p = _sc_gather_add(hso, hdo, src, dst, xr)
    s_src, s_dst, c_src, c_dst = _sc_scatter(p_rs, p_rd, ssrc, sdst)

    h1p = _combine(s_src, s_dst, c_src, c_dst, hr, br=2000)

    f1s, f1ss = _stats(f1p, br=2000)
    h1s, h1ss = _stats(h1p, br=2000)

    xr2 = _bn_mm(f1p, f1s, f1ss, n1f_g, n1f_b, c2_so_Wr + c2_do_Wr,
                 c2_so_bl + c2_do_bl, br=2000)
    ga = _bn_mm(h1p, h1s, h1ss, n1h_g, n1h_b, c2_so_Wl,
                jnp.zeros((HID,), jnp.float32), br=2000)
    gb = _bn_mm(h1p, h1s, h1ss, n1h_g, n1h_b, c2_do_Wl,
                jnp.zeros((HID,), jnp.float32), br=2000)

    f2p = _sc_gather_add(ga, gb, src, dst, xr2)

    f2s, f2ss = _stats(f2p, br=2000)
    return _bn_mm(f2p, f2s, f2ss, n2f_g, n2f_b, lin_W, lin_b, br=2000)


# speedup vs baseline: 3.1169x; 1.1812x over previous
import functools

import jax
import jax.numpy as jnp
from jax import lax
from jax.experimental import pallas as pl
from jax.experimental.pallas import tpu as pltpu
import jax.experimental.pallas.tpu_sc as plsc

NF = 100000
NH = 50000
DF = 128
EMB = 32
HID = 64
OUT = 2
EPS = 1e-5

NCORE = 2
NSUB = 16
NW = NCORE * NSUB
CH = 80
NCH = NF // CH
HW = HID // NCORE
ZCH = 128


@functools.cache
def _mesh():
    return plsc.VectorSubcoreMesh(core_axis_name="c", subcore_axis_name="s",
                                  num_cores=NCORE, num_subcores=NSUB)


def _mm3_body(x_ref, w1_ref, w2_ref, w3_ref, b3_ref, o1_ref, o2_ref, o3_ref):
    x = x_ref[...]
    o1_ref[...] = jnp.dot(x, w1_ref[...], preferred_element_type=jnp.float32)
    o2_ref[...] = jnp.dot(x, w2_ref[...], preferred_element_type=jnp.float32)
    o3_ref[...] = (jnp.dot(x, w3_ref[...], preferred_element_type=jnp.float32)
                   + b3_ref[...])


def _mm3(x, w1, w2, w3, b3, br):
    n, d = x.shape
    return pl.pallas_call(
        _mm3_body,
        grid=(n // br,),
        in_specs=[
            pl.BlockSpec((br, d), lambda i: (i, 0)),
            pl.BlockSpec((d, HID), lambda i: (0, 0)),
            pl.BlockSpec((d, HID), lambda i: (0, 0)),
            pl.BlockSpec((d, HID), lambda i: (0, 0)),
            pl.BlockSpec((1, HID), lambda i: (0, 0)),
        ],
        out_specs=[
            pl.BlockSpec((br, HID), lambda i: (i, 0)),
            pl.BlockSpec((br, HID), lambda i: (i, 0)),
            pl.BlockSpec((br, HID), lambda i: (i, 0)),
        ],
        out_shape=[jax.ShapeDtypeStruct((n, HID), jnp.float32)] * 3,
    )(x, w1, w2, w3, b3.reshape(1, HID))


def _stats_body(x_ref, s_ref, ss_ref):
    x = x_ref[...]
    ps = jnp.sum(x, axis=0, keepdims=True)
    pss = jnp.sum(x * x, axis=0, keepdims=True)

    @pl.when(pl.program_id(0) == 0)
    def _():
        s_ref[...] = ps
        ss_ref[...] = pss

    @pl.when(pl.program_id(0) != 0)
    def _():
        s_ref[...] += ps
        ss_ref[...] += pss


def _stats(x, br):
    n, _ = x.shape
    return pl.pallas_call(
        _stats_body,
        grid=(n // br,),
        in_specs=[pl.BlockSpec((br, HID), lambda i: (i, 0))],
        out_specs=[pl.BlockSpec((1, HID), lambda i: (0, 0))] * 2,
        out_shape=[jax.ShapeDtypeStruct((1, HID), jnp.float32)] * 2,
    )(x)


def _combine_stats_body(ssrc_ref, sdst_ref, csrc_ref, cdst_ref, hr_ref,
                        o_ref, s_ref, ss_ref):
    c1 = jnp.maximum(csrc_ref[...][:, 0:1], 1.0)
    c2 = jnp.maximum(cdst_ref[...][:, 0:1], 1.0)
    h = ssrc_ref[...] / c1 + sdst_ref[...] / c2 + hr_ref[...]
    o_ref[...] = h
    ps = jnp.sum(h, axis=0, keepdims=True)
    pss = jnp.sum(h * h, axis=0, keepdims=True)

    @pl.when(pl.program_id(0) == 0)
    def _():
        s_ref[...] = ps
        ss_ref[...] = pss

    @pl.when(pl.program_id(0) != 0)
    def _():
        s_ref[...] += ps
        ss_ref[...] += pss


def _combine_stats(ssrc, sdst, csrc, cdst, hr, br):
    n, _ = ssrc.shape
    return pl.pallas_call(
        _combine_stats_body,
        grid=(n // br,),
        in_specs=[
            pl.BlockSpec((br, HID), lambda i: (i, 0)),
            pl.BlockSpec((br, HID), lambda i: (i, 0)),
            pl.BlockSpec((br, 16), lambda i: (i, 0)),
            pl.BlockSpec((br, 16), lambda i: (i, 0)),
            pl.BlockSpec((br, HID), lambda i: (i, 0)),
        ],
        out_specs=[
            pl.BlockSpec((br, HID), lambda i: (i, 0)),
            pl.BlockSpec((1, HID), lambda i: (0, 0)),
            pl.BlockSpec((1, HID), lambda i: (0, 0)),
        ],
        out_shape=[
            jax.ShapeDtypeStruct((n, HID), jnp.float32),
            jax.ShapeDtypeStruct((1, HID), jnp.float32),
            jax.ShapeDtypeStruct((1, HID), jnp.float32),
        ],
    )(ssrc, sdst, csrc, cdst, hr)


def _make_bn_mm_body(n):
    def body(x_ref, s_ref, ss_ref, g_ref, b_ref, w_ref, bias_ref, o_ref):
        m = s_ref[...] / n
        v = ss_ref[...] / n - m * m
        scale = g_ref[...] * lax.rsqrt(v + EPS)
        shift = b_ref[...] - m * scale
        h = jnp.maximum(x_ref[...] * scale + shift, 0.0)
        o_ref[...] = (jnp.dot(h, w_ref[...], preferred_element_type=jnp.float32)
                      + bias_ref[...])
    return body


def _bn_mm(x, s, ss, g, b, w, bias, br):
    n, _ = x.shape
    k = w.shape[1]
    return pl.pallas_call(
        _make_bn_mm_body(float(n)),
        grid=(n // br,),
        in_specs=[
            pl.BlockSpec((br, HID), lambda i: (i, 0)),
            pl.BlockSpec((1, HID), lambda i: (0, 0)),
            pl.BlockSpec((1, HID), lambda i: (0, 0)),
            pl.BlockSpec((1, HID), lambda i: (0, 0)),
            pl.BlockSpec((1, HID), lambda i: (0, 0)),
            pl.BlockSpec((HID, k), lambda i: (0, 0)),
            pl.BlockSpec((1, k), lambda i: (0, 0)),
        ],
        out_specs=pl.BlockSpec((br, k), lambda i: (i, 0)),
        out_shape=jax.ShapeDtypeStruct((n, k), jnp.float32),
    )(x, s, ss, g.reshape(1, HID), b.reshape(1, HID), w, bias.reshape(1, k))


def _make_bn_mm2_body(n):
    def body(x_ref, s_ref, ss_ref, g_ref, b_ref, w1_ref, w2_ref,
             o1_ref, o2_ref):
        m = s_ref[...] / n
        v = ss_ref[...] / n - m * m
        scale = g_ref[...] * lax.rsqrt(v + EPS)
        shift = b_ref[...] - m * scale
        h = jnp.maximum(x_ref[...] * scale + shift, 0.0)
        o1_ref[...] = jnp.dot(h, w1_ref[...], preferred_element_type=jnp.float32)
        o2_ref[...] = jnp.dot(h, w2_ref[...], preferred_element_type=jnp.float32)
    return body


def _bn_mm2(x, s, ss, g, b, w1, w2, br):
    n, _ = x.shape
    return pl.pallas_call(
        _make_bn_mm2_body(float(n)),
        grid=(n // br,),
        in_specs=[
            pl.BlockSpec((br, HID), lambda i: (i, 0)),
            pl.BlockSpec((1, HID), lambda i: (0, 0)),
            pl.BlockSpec((1, HID), lambda i: (0, 0)),
            pl.BlockSpec((1, HID), lambda i: (0, 0)),
            pl.BlockSpec((1, HID), lambda i: (0, 0)),
            pl.BlockSpec((HID, HID), lambda i: (0, 0)),
            pl.BlockSpec((HID, HID), lambda i: (0, 0)),
        ],
        out_specs=[
            pl.BlockSpec((br, HID), lambda i: (i, 0)),
            pl.BlockSpec((br, HID), lambda i: (i, 0)),
        ],
        out_shape=[jax.ShapeDtypeStruct((n, HID), jnp.float32)] * 2,
    )(x, s, ss, g.reshape(1, HID), b.reshape(1, HID), w1, w2)


def _gather_pipeline(ta_h, tb_h, ia_h, ib_h, xc_h, out_h, bufs):
    wid = lax.axis_index("s") * NCORE + lax.axis_index("c")
    nj = (NCH + NW - 1) // NW

    def issue(c, buf):
        ia_v, ib_v, ra_v, rb_v, rc_v, sem_a, sem_b, sem_c = buf
        base = c * CH
        pltpu.sync_copy(ia_h.at[pl.ds(base, CH)], ia_v)
        pltpu.sync_copy(ib_h.at[pl.ds(base, CH)], ib_v)
        pltpu.async_copy(ta_h.at[ia_v], ra_v, sem_a)
        pltpu.async_copy(tb_h.at[ib_v], rb_v, sem_b)
        pltpu.async_copy(xc_h.at[pl.ds(base, CH)], rc_v, sem_c)

    def consume(c, buf):
        ia_v, ib_v, ra_v, rb_v, rc_v, sem_a, sem_b, sem_c = buf
        base = c * CH
        pltpu.make_async_copy(ta_h.at[ia_v], ra_v, sem_a).wait()
        pltpu.make_async_copy(tb_h.at[ib_v], rb_v, sem_b).wait()
        pltpu.make_async_copy(xc_h.at[pl.ds(base, CH)], rc_v, sem_c).wait()

        def add_g(g, carry2):
            r = g // (HID // 16)
            col = (g % (HID // 16)) * 16
            ra_v[r, pl.ds(col, 16)] = (ra_v[r, pl.ds(col, 16)]
                                       + rb_v[r, pl.ds(col, 16)]
                                       + rc_v[r, pl.ds(col, 16)])
            return carry2

        lax.fori_loop(0, CH * (HID // 16), add_g, 0, unroll=8)
        pltpu.sync_copy(ra_v, out_h.at[pl.ds(base, CH)])

    @pl.when(wid < NCH)
    def _():
        issue(wid, bufs[0])

    def pair(j2, carry):
        for b in (0, 1):
            j = 2 * j2 + b
            c = wid + NW * j
            cn = wid + NW * (j + 1)

            @pl.when(cn < NCH)
            def _():
                issue(cn, bufs[1 - b])

            @pl.when(c < NCH)
            def _():
                consume(c, bufs[b])

        return carry

    lax.fori_loop(0, (nj + 1) // 2, pair, 0)


_GATHER_SCRATCH = (
    [pltpu.VMEM((CH,), jnp.int32)] * 2
    + [pltpu.VMEM((CH, HID), jnp.float32)] * 3
    + [pltpu.SemaphoreType.DMA] * 3
    + [pltpu.VMEM((CH,), jnp.int32)] * 2
    + [pltpu.VMEM((CH, HID), jnp.float32)] * 3
    + [pltpu.SemaphoreType.DMA] * 3
)


def _sc_gather_add_body(ta_h, tb_h, ia_h, ib_h, xc_h, out_h, *scratch):
    bufs = (scratch[0:8], scratch[8:16])
    _gather_pipeline(ta_h, tb_h, ia_h, ib_h, xc_h, out_h, bufs)


def _sc_gather_add(ta, tb, ia, ib, xc):
    return pl.kernel(
        _sc_gather_add_body,
        out_type=jax.ShapeDtypeStruct((NF, HID), jnp.float32),
        mesh=_mesh(),
        compiler_params=pltpu.CompilerParams(use_tc_tiling_on_sc=False),
        scratch_types=list(_GATHER_SCRATCH),
    )(ta, tb, ia, ib, xc)


def _sc_gather_counts_body(ta_h, tb_h, ia_h, ib_h, xc_h, ica_h, icb_h,
                           out_h, ca_h, cb_h, *scratch):
    gbufs = (scratch[0:8], scratch[8:16])
    cidx0, cidx1, ones_v, zcnt_v, sci0, sci1, cnt_sh = scratch[16:23]
    cid = lax.axis_index("c")
    sid = lax.axis_index("s")
    ones16 = jnp.ones((16,), jnp.float32)
    zero16 = jnp.zeros((16,), jnp.float32)

    def init_ones(r, carry):
        ones_v[r, :] = ones16
        return carry

    lax.fori_loop(0, CH, init_ones, 0)

    def init_zcnt(r, carry):
        zcnt_v[r, :] = zero16
        return carry

    lax.fori_loop(0, ZCH, init_zcnt, 0)

    nzc = (NH + ZCH - 1) // ZCH

    def zero_chunk(j, carry):
        c = sid + NSUB * j

        @pl.when(c < nzc)
        def _():
            b = jnp.minimum(c * ZCH, NH - ZCH)
            pltpu.sync_copy(zcnt_v, cnt_sh.at[pl.ds(b, ZCH)])

        return carry

    lax.fori_loop(0, (nzc + NSUB - 1) // NSUB, zero_chunk, 0)
    plsc.subcore_barrier()

    _gather_pipeline(ta_h, tb_h, ia_h, ib_h, xc_h, out_h, gbufs)

    cbufs = ((cidx0, sci0), (cidx1, sci1))

    def run_counts(i_h):
        nj = (NCH + NSUB - 1) // NSUB

        def issue(c, buf):
            idx_v, sem_i = buf
            pltpu.async_copy(i_h.at[pl.ds(c * CH, CH)], idx_v, sem_i)

        def consume(c, buf):
            idx_v, sem_i = buf
            pltpu.make_async_copy(i_h.at[pl.ds(c * CH, CH)], idx_v, sem_i).wait()
            pltpu.sync_copy(ones_v, cnt_sh.at[idx_v], add=True)

        @pl.when(sid < NCH)
        def _():
            issue(sid, cbufs[0])

        def pair(j2, carry):
            for b in (0, 1):
                j = 2 * j2 + b
                c = sid + NSUB * j
                cn = sid + NSUB * (j + 1)

                @pl.when(cn < NCH)
                def _():
                    issue(cn, cbufs[1 - b])

                @pl.when(c < NCH)
                def _():
                    consume(c, cbufs[b])

            return carry

        lax.fori_loop(0, (nj + 1) // 2, pair, 0)

    @pl.when(cid == 0)
    def _():
        run_counts(ica_h)

    @pl.when(cid == 1)
    def _():
        run_counts(icb_h)

    plsc.subcore_barrier()

    def flush(c_h):
        def flush_chunk(j, carry):
            c = sid + NSUB * j

            @pl.when(c < nzc)
            def _():
                b = jnp.minimum(c * ZCH, NH - ZCH)
                pltpu.sync_copy(cnt_sh.at[pl.ds(b, ZCH)], c_h.at[pl.ds(b, ZCH)])

            return carry

        lax.fori_loop(0, (nzc + NSUB - 1) // NSUB, flush_chunk, 0)

    @pl.when(cid == 0)
    def _():
        flush(ca_h)

    @pl.when(cid == 1)
    def _():
        flush(cb_h)

    plsc.subcore_barrier()


def _sc_gather_counts(ta, tb, ia, ib, xc, ica, icb):
    return pl.kernel(
        _sc_gather_counts_body,
        out_type=[
            jax.ShapeDtypeStruct((NF, HID), jnp.float32),
            jax.ShapeDtypeStruct((NH, 16), jnp.float32),
            jax.ShapeDtypeStruct((NH, 16), jnp.float32),
        ],
        mesh=_mesh(),
        compiler_params=pltpu.CompilerParams(use_tc_tiling_on_sc=False),
        scratch_types=list(_GATHER_SCRATCH) + [
            pltpu.VMEM((CH,), jnp.int32),
            pltpu.VMEM((CH,), jnp.int32),
            pltpu.VMEM((CH, 16), jnp.float32),
            pltpu.VMEM((ZCH, 16), jnp.float32),
            pltpu.SemaphoreType.DMA,
            pltpu.SemaphoreType.DMA,
            pltpu.VMEM_SHARED((NH, 16), jnp.float32),
        ],
    )(ta, tb, ia, ib, xc, ica, icb)


def _sc_sums_body(pa_h, pb_h, ia_h, ib_h, sa_h, sb_h,
                  idx0, idx1, val0, val1, zrow_v,
                  si0, si1, sv0, sv1, ssum_sh):
    cid = lax.axis_index("c")
    sid = lax.axis_index("s")
    col0 = cid * HW
    zero16 = jnp.zeros((16,), jnp.float32)

    def init_zero(g, carry):
        zrow_v[g // (HW // 16), pl.ds((g % (HW // 16)) * 16, 16)] = zero16
        return carry

    lax.fori_loop(0, ZCH * (HW // 16), init_zero, 0)

    nzc = (NH + ZCH - 1) // ZCH
    nj = (NCH + NSUB - 1) // NSUB
    bufs = ((idx0, val0, si0, sv0), (idx1, val1, si1, sv1))

    for p_h, i_h, s_h in ((pa_h, ia_h, sa_h), (pb_h, ib_h, sb_h)):
        def zero_chunk(j, carry):
            c = sid + NSUB * j

            @pl.when(c < nzc)
            def _():
                b = jnp.minimum(c * ZCH, NH - ZCH)
                pltpu.sync_copy(zrow_v, ssum_sh.at[pl.ds(b, ZCH)])

            return carry

        lax.fori_loop(0, (nzc + NSUB - 1) // NSUB, zero_chunk, 0)
        plsc.subcore_barrier()

        def issue(c, buf, i_h=i_h, p_h=p_h):
            idx_v, val_v, sem_i, sem_v = buf
            base = c * CH
            pltpu.async_copy(i_h.at[pl.ds(base, CH)], idx_v, sem_i)
            pltpu.async_copy(p_h.at[pl.ds(base, CH), pl.ds(col0, HW)],
                             val_v, sem_v)

        def consume(c, buf, i_h=i_h, p_h=p_h):
            idx_v, val_v, sem_i, sem_v = buf
            base = c * CH
            pltpu.make_async_copy(i_h.at[pl.ds(base, CH)], idx_v, sem_i).wait()
            pltpu.make_async_copy(p_h.at[pl.ds(base, CH), pl.ds(col0, HW)],
                                  val_v, sem_v).wait()
            pltpu.sync_copy(val_v, ssum_sh.at[idx_v], add=True)

        @pl.when(sid < NCH)
        def _():
            issue(sid, bufs[0])

        def pair(j2, carry, issue=issue, consume=consume):
            for b in (0, 1):
                j = 2 * j2 + b
                c = sid + NSUB * j
                cn = sid + NSUB * (j + 1)

                @pl.when(cn < NCH)
                def _():
                    issue(cn, bufs[1 - b])

                @pl.when(c < NCH)
                def _():
                    consume(c, bufs[b])

            return carry

        lax.fori_loop(0, (nj + 1) // 2, pair, 0)
        plsc.subcore_barrier()

        def flush_chunk(j, carry, s_h=s_h):
            c = sid + NSUB * j

            @pl.when(c < nzc)
            def _():
                b = jnp.minimum(c * ZCH, NH - ZCH)
                pltpu.sync_copy(ssum_sh.at[pl.ds(b, ZCH)],
                                s_h.at[pl.ds(b, ZCH), pl.ds(col0, HW)])

            return carry

        lax.fori_loop(0, (nzc + NSUB - 1) // NSUB, flush_chunk, 0)
        plsc.subcore_barrier()


def _sc_sums(pa, pb, ia, ib):
    return pl.kernel(
        _sc_sums_body,
        out_type=[
            jax.ShapeDtypeStruct((NH, HID), jnp.float32),
            jax.ShapeDtypeStruct((NH, HID), jnp.float32),
        ],
        mesh=_mesh(),
        compiler_params=pltpu.CompilerParams(use_tc_tiling_on_sc=False),
        scratch_types=[
            pltpu.VMEM((CH,), jnp.int32),
            pltpu.VMEM((CH,), jnp.int32),
            pltpu.VMEM((CH, HW), jnp.float32),
            pltpu.VMEM((CH, HW), jnp.float32),
            pltpu.VMEM((ZCH, HW), jnp.float32),
            pltpu.SemaphoreType.DMA,
            pltpu.SemaphoreType.DMA,
            pltpu.SemaphoreType.DMA,
            pltpu.SemaphoreType.DMA,
            pltpu.VMEM_SHARED((NH, HW), jnp.float32),
        ],
    )(pa, pb, ia, ib)


def kernel(x_flow, ei_src_of, ei_dst_of, ei_rev_src_of, ei_rev_dst_of, host_emb,
           c1_so_Wl, c1_so_bl, c1_so_Wr, c1_do_Wl, c1_do_bl, c1_do_Wr,
           c1_rs_Wl, c1_rs_bl, c1_rs_Wr, c1_rd_Wl, c1_rd_bl, c1_rd_Wr,
           c2_so_Wl, c2_so_bl, c2_so_Wr, c2_do_Wl, c2_do_bl, c2_do_Wr,
           c2_rs_Wl, c2_rs_bl, c2_rs_Wr, c2_rd_Wl, c2_rd_bl, c2_rd_Wr,
           n1f_g, n1f_b, n1h_g, n1h_b, n2f_g, n2f_b, n2h_g, n2h_b,
           lin_W, lin_b):
    src = ei_src_of[0]
    dst = ei_dst_of[0]
    ssrc = ei_rev_src_of[1]
    sdst = ei_rev_dst_of[1]

    p_rs, p_rd, xr = _mm3(x_flow, c1_rs_Wl, c1_rd_Wl, c1_so_Wr + c1_do_Wr,
                          c1_so_bl + c1_do_bl, br=2000)
    hso, hdo, hr = _mm3(host_emb, c1_so_Wl, c1_do_Wl, c1_rs_Wr + c1_rd_Wr,
                        c1_rs_bl + c1_rd_bl, br=2000)

    f1p, c_src, c_dst = _sc_gather_counts(hso, hdo, src, dst, xr, ssrc, sdst)
    s_src, s_dst = _sc_sums(p_rs, p_rd, ssrc, sdst)

    h1p, h1s, h1ss = _combine_stats(s_src, s_dst, c_src, c_dst, hr, br=2000)
    f1s, f1ss = _stats(f1p, br=2000)

    xr2 = _bn_mm(f1p, f1s, f1ss, n1f_g, n1f_b, c2_so_Wr + c2_do_Wr,
                 c2_so_bl + c2_do_bl, br=2000)
    ga, gb = _bn_mm2(h1p, h1s, h1ss, n1h_g, n1h_b, c2_so_Wl, c2_do_Wl, br=2000)

    f2p = _sc_gather_add(ga, gb, src, dst, xr2)

    f2s, f2ss = _stats(f2p, br=2000)
    return _bn_mm(f2p, f2s, f2ss, n2f_g, n2f_b, lin_W, lin_b, br=2000)
